# baseline (speedup 1.0000x reference)
import functools

import jax
import jax.numpy as jnp
from jax import lax
from jax.experimental import pallas as pl
from jax.experimental.pallas import tpu as pltpu
from jax.experimental.pallas import tpu_sc as plsc

_NC = 2
_NS = 16
_NW = _NC * _NS
_CH = 128
_NBUF = 4
_DP = 128


def _pe_table(max_len, d_embed):
    pos = jnp.arange(max_len, dtype=jnp.float32)[:, None]
    i = jnp.arange(0, d_embed, 2, dtype=jnp.float32)[None, :]
    angle = pos / jnp.power(10000.0, i / d_embed)
    pe = jnp.zeros((max_len, d_embed), dtype=jnp.float32)
    pe = pe.at[:, 0::2].set(jnp.sin(angle))
    pe = pe.at[:, 1::2].set(jnp.cos(angle))
    return pe


def _prep_table(tok_t, tok_tail):
    D, V = tok_t.shape
    n_full = V // 128
    tail = V - n_full * 128
    n_blocks = n_full + (1 if tail else 0)
    per_w = (n_blocks + _NW - 1) // _NW
    vp2 = (n_full + 1) * 64
    mesh = plsc.VectorSubcoreMesh(core_axis_name="c", subcore_axis_name="s")

    @functools.partial(
        pl.kernel,
        out_type=jax.ShapeDtypeStruct((vp2, 2 * D), jnp.float32),
        mesh=mesh,
        scratch_types=[
            pltpu.VMEM((D, 128), jnp.float32),
            pltpu.VMEM((64, 2 * D), jnp.float32),
            pltpu.SemaphoreType.DMA,
        ],
        compiler_params=pltpu.CompilerParams(needs_layout_passes=False),
    )
    def prep(tt_hbm, ttail_hbm, out_hbm, ibuf, obuf, isem):
        wid = lax.axis_index("s") * _NC + lax.axis_index("c")

        dvs = [lax.iota(jnp.int32, 16) + (16 * k) % D for k in range(2 * D // 16)]

        def transpose_block():
            @plsc.parallel_loop(0, 64, 1, unroll=4)
            def rows(j):
                for h in range(2):
                    iv = jnp.zeros((16,), jnp.int32) + (2 * j + h)
                    for k4 in range(D // 16):
                        k = h * (D // 16) + k4
                        obuf[j, pl.ds(16 * k, 16)] = plsc.load_gather(
                            ibuf, [dvs[k], iv]
                        )

        def step(g, _):
            c = wid * per_w + g

            @pl.when(c < n_full)
            def _():
                pltpu.async_copy(
                    tt_hbm.at[:, pl.ds(c * 128, 128)], ibuf, isem
                ).wait()
                transpose_block()
                pltpu.sync_copy(obuf, out_hbm.at[pl.ds(c * 64, 64)])

            @pl.when(c == n_full)
            def _():
                pltpu.async_copy(ttail_hbm, ibuf, isem).wait()
                transpose_block()
                pltpu.sync_copy(obuf, out_hbm.at[pl.ds(c * 64, 64)])

            return 0

        lax.fori_loop(0, per_w, step, 0)

    return prep(tok_t, tok_tail)


@functools.partial(jax.jit, static_argnums=(3, 4, 5))
def _emb_call(tok_lin, idx, pe_ext, N, D, S):
    n_per_w = N // _NW
    n_chunks = n_per_w // _CH
    assert n_chunks % _NBUF == 0
    pe_rows = pe_ext.shape[0]
    mesh = plsc.VectorSubcoreMesh(core_axis_name="c", subcore_axis_name="s")

    @functools.partial(
        pl.kernel,
        out_type=jax.ShapeDtypeStruct((N, _DP), jnp.float32),
        mesh=mesh,
        scratch_types=[
            pltpu.VMEM((n_per_w,), jnp.int32),
            [pltpu.VMEM((_CH, D), jnp.float32)] * _NBUF,
            pltpu.VMEM((pe_rows, D), jnp.float32),
            [pltpu.SemaphoreType.DMA] * _NBUF,
            [pltpu.SemaphoreType.DMA] * _NBUF,
        ],
        compiler_params=pltpu.CompilerParams(use_tc_tiling_on_sc=False),
    )
    def emb(tok_hbm, idx_hbm, pe_hbm, out_hbm, idx_v, bufs, pe_v, gsems, ssems):
        wid = lax.axis_index("s") * _NC + lax.axis_index("c")
        base = wid * n_per_w
        pltpu.sync_copy(idx_hbm.at[pl.ds(base, n_per_w)], idx_v)
        pltpu.sync_copy(pe_hbm, pe_v)

        def fire_gather(g, bi):
            pltpu.async_copy(
                tok_hbm.at[idx_v.at[pl.ds(g * _CH, _CH)]], bufs[bi], gsems[bi]
            )

        def wait_gather(bi):
            pltpu.make_async_copy(
                tok_hbm.at[idx_v.at[pl.ds(0, _CH)]], bufs[bi], gsems[bi]
            ).wait()

        def wait_store(bi):
            pltpu.make_async_copy(
                bufs[bi], out_hbm.at[pl.ds(0, _CH), pl.ds(0, D)], ssems[bi]
            ).wait()

        fire_gather(0, 0)
        fire_gather(1, 1)

        def process(g, bi):
            wait_gather(bi)
            start = lax.rem(base + g * _CH, S)
            buf = bufs[bi]

            @plsc.parallel_loop(0, _CH, 1, unroll=8)
            def radd(r):
                p = start + r
                for k in range(D // 16):
                    sl = pl.ds(k * 16, 16)
                    buf[r, sl] = buf[r, sl] + pe_v[p, sl]

            pltpu.async_copy(
                buf,
                out_hbm.at[pl.ds(base + g * _CH, _CH), pl.ds(0, D)],
                ssems[bi],
            )

            nb = (bi + 2) % _NBUF

            @pl.when(g >= 2)
            def _():
                wait_store(nb)

            @pl.when(g + 2 < n_chunks)
            def _():
                fire_gather(g + 2, nb)

        def group(t, _):
            for b in range(_NBUF):
                process(t * _NBUF + b, b)
            return 0

        lax.fori_loop(0, n_chunks // _NBUF, group, 0)
        wait_store(_NBUF - 2)
        wait_store(_NBUF - 1)

    return emb(tok_lin, idx, pe_ext)


def kernel(x, tok_table):
    B, S = x.shape
    V, D = tok_table.shape
    N = B * S
    idx = x.reshape(N).astype(jnp.int32)
    pe = _pe_table(S, D)
    pe_ext = jnp.concatenate([pe, pe[:_CH]], axis=0)
    n_full = V // 128
    tail = V - n_full * 128
    tok_tail = jnp.pad(
        tok_table.T[:, n_full * 128:], ((0, 0), (0, 128 - tail))
    )
    tok_compact = _prep_table(tok_table.T, tok_tail)
    tok_lin = tok_compact.reshape(tok_compact.shape[0] * 2, D)
    out = _emb_call(tok_lin, idx, pe_ext, N, D, S)
    return out.reshape(B, S, _DP)[:, :, :D]


# speedup vs baseline: 1.2988x; 1.2988x over previous
import functools

import jax
import jax.numpy as jnp
from jax import lax
from jax.experimental import pallas as pl
from jax.experimental.pallas import tpu as pltpu
from jax.experimental.pallas import tpu_sc as plsc

_NC = 2
_NS = 16
_NW = _NC * _NS
_CH = 128
_NBUF = 4
_DP = 128


def _pe_table(max_len, d_embed):
    pos = jnp.arange(max_len, dtype=jnp.float32)[:, None]
    i = jnp.arange(0, d_embed, 2, dtype=jnp.float32)[None, :]
    angle = pos / jnp.power(10000.0, i / d_embed)
    pe = jnp.zeros((max_len, d_embed), dtype=jnp.float32)
    pe = pe.at[:, 0::2].set(jnp.sin(angle))
    pe = pe.at[:, 1::2].set(jnp.cos(angle))
    return pe


def _prep_table(tok_t, tok_tail):
    D, V = tok_t.shape
    n_full = V // 128
    tail = V - n_full * 128
    n_blocks = n_full + (1 if tail else 0)
    per_w = (n_blocks + _NW - 1) // _NW
    vp2 = (n_full + 1) * 64
    mesh = plsc.VectorSubcoreMesh(core_axis_name="c", subcore_axis_name="s")

    _PB = 4

    @functools.partial(
        pl.kernel,
        out_type=jax.ShapeDtypeStruct((vp2, 2 * D), jnp.float32),
        mesh=mesh,
        scratch_types=[
            [pltpu.VMEM((D, 128), jnp.float32)] * _PB,
            [pltpu.VMEM((64, 2 * D), jnp.float32)] * _PB,
            [pltpu.SemaphoreType.DMA] * _PB,
            [pltpu.SemaphoreType.DMA] * _PB,
        ],
        compiler_params=pltpu.CompilerParams(needs_layout_passes=False),
    )
    def prep(tt_hbm, ttail_hbm, out_hbm, ibufs, obufs, isems, osems):
        wid = lax.axis_index("s") * _NC + lax.axis_index("c")

        def cid(g):
            return wid * per_w + g

        def valid(g):
            return (g >= 0) & (g < per_w) & (cid(g) < n_blocks)

        def fire_load(g, bi):
            @pl.when(valid(g))
            def _():
                c = cid(g)

                @pl.when(c < n_full)
                def _():
                    pltpu.async_copy(
                        tt_hbm.at[:, pl.ds(c * 128, 128)], ibufs[bi], isems[bi]
                    )

                @pl.when(c == n_full)
                def _():
                    pltpu.async_copy(ttail_hbm, ibufs[bi], isems[bi])

        def wait_load(bi):
            pltpu.make_async_copy(
                tt_hbm.at[:, pl.ds(0, 128)], ibufs[bi], isems[bi]
            ).wait()

        def wait_store(bi):
            pltpu.make_async_copy(
                obufs[bi], out_hbm.at[pl.ds(0, 64)], osems[bi]
            ).wait()

        dvs = [lax.iota(jnp.int32, 16) + (16 * k) % D for k in range(2 * D // 16)]

        def transpose_block(bi):
            ib = ibufs[bi]
            ob = obufs[bi]

            @plsc.parallel_loop(0, 64, 1, unroll=4)
            def rows(j):
                for h in range(2):
                    iv = jnp.zeros((16,), jnp.int32) + (2 * j + h)
                    for k4 in range(D // 16):
                        k = h * (D // 16) + k4
                        ob[j, pl.ds(16 * k, 16)] = plsc.load_gather(
                            ib, [dvs[k], iv]
                        )

        fire_load(0, 0)
        fire_load(1, 1)

        def step(g, bi):
            @pl.when(valid(g - _PB))
            def _():
                wait_store(bi)

            @pl.when(valid(g))
            def _():
                wait_load(bi)
                transpose_block(bi)
                pltpu.async_copy(
                    obufs[bi], out_hbm.at[pl.ds(cid(g) * 64, 64)], osems[bi]
                )

            fire_load(g + 2, (bi + 2) % _PB)

        def group(t, _):
            for b in range(_PB):
                step(t * _PB + b, b)
            return 0

        n_groups = (per_w + _PB - 1) // _PB
        lax.fori_loop(0, n_groups, group, 0)
        for g_e in range(n_groups * _PB - _PB, n_groups * _PB):
            @pl.when(valid(g_e))
            def _(g_e=g_e):
                wait_store(g_e % _PB)

    return prep(tok_t, tok_tail)


@functools.partial(jax.jit, static_argnums=(3, 4, 5))
def _emb_call(tok_lin, idx, pe_ext, N, D, S):
    n_per_w = N // _NW
    n_chunks = n_per_w // _CH
    assert n_chunks % _NBUF == 0
    pe_rows = pe_ext.shape[0]
    mesh = plsc.VectorSubcoreMesh(core_axis_name="c", subcore_axis_name="s")

    @functools.partial(
        pl.kernel,
        out_type=jax.ShapeDtypeStruct((N, _DP), jnp.float32),
        mesh=mesh,
        scratch_types=[
            pltpu.VMEM((n_per_w,), jnp.int32),
            [pltpu.VMEM((_CH, D), jnp.float32)] * _NBUF,
            pltpu.VMEM((pe_rows, D), jnp.float32),
            [pltpu.SemaphoreType.DMA] * _NBUF,
            [pltpu.SemaphoreType.DMA] * _NBUF,
        ],
        compiler_params=pltpu.CompilerParams(use_tc_tiling_on_sc=False),
    )
    def emb(tok_hbm, idx_hbm, pe_hbm, out_hbm, idx_v, bufs, pe_v, gsems, ssems):
        wid = lax.axis_index("s") * _NC + lax.axis_index("c")
        base = wid * n_per_w
        pltpu.sync_copy(idx_hbm.at[pl.ds(base, n_per_w)], idx_v)
        pltpu.sync_copy(pe_hbm, pe_v)

        def fire_gather(g, bi):
            pltpu.async_copy(
                tok_hbm.at[idx_v.at[pl.ds(g * _CH, _CH)]], bufs[bi], gsems[bi]
            )

        def wait_gather(bi):
            pltpu.make_async_copy(
                tok_hbm.at[idx_v.at[pl.ds(0, _CH)]], bufs[bi], gsems[bi]
            ).wait()

        def wait_store(bi):
            pltpu.make_async_copy(
                bufs[bi], out_hbm.at[pl.ds(0, _CH), pl.ds(0, D)], ssems[bi]
            ).wait()

        fire_gather(0, 0)
        fire_gather(1, 1)

        def process(g, bi):
            wait_gather(bi)
            start = lax.rem(base + g * _CH, S)
            buf = bufs[bi]

            @plsc.parallel_loop(0, _CH, 1, unroll=8)
            def radd(r):
                p = start + r
                for k in range(D // 16):
                    sl = pl.ds(k * 16, 16)
                    buf[r, sl] = buf[r, sl] + pe_v[p, sl]

            pltpu.async_copy(
                buf,
                out_hbm.at[pl.ds(base + g * _CH, _CH), pl.ds(0, D)],
                ssems[bi],
            )

            nb = (bi + 2) % _NBUF

            @pl.when(g >= 2)
            def _():
                wait_store(nb)

            @pl.when(g + 2 < n_chunks)
            def _():
                fire_gather(g + 2, nb)

        def group(t, _):
            for b in range(_NBUF):
                process(t * _NBUF + b, b)
            return 0

        lax.fori_loop(0, n_chunks // _NBUF, group, 0)
        wait_store(_NBUF - 2)
        wait_store(_NBUF - 1)

    return emb(tok_lin, idx, pe_ext)


def kernel(x, tok_table):
    B, S = x.shape
    V, D = tok_table.shape
    N = B * S
    idx = x.reshape(N).astype(jnp.int32)
    pe = _pe_table(S, D)
    pe_ext = jnp.concatenate([pe, pe[:_CH]], axis=0)
    n_full = V // 128
    tail = V - n_full * 128
    tok_tail = jnp.pad(
        tok_table.T[:, n_full * 128:], ((0, 0), (0, 128 - tail))
    )
    tok_compact = _prep_table(tok_table.T, tok_tail)
    tok_lin = tok_compact.reshape(tok_compact.shape[0] * 2, D)
    out = _emb_call(tok_lin, idx, pe_ext, N, D, S)
    return out.reshape(B, S, _DP)[:, :, :D]


# speedup vs baseline: 1.3274x; 1.0220x over previous
import functools

import jax
import jax.numpy as jnp
from jax import lax
from jax.experimental import pallas as pl
from jax.experimental.pallas import tpu as pltpu
from jax.experimental.pallas import tpu_sc as plsc

_NC = 2
_NS = 16
_NW = _NC * _NS
_CH = 128
_NBUF = 4
_DP = 128


_TBLK = 1024


def _prep_table_tc(tok_t, eye):
    D, V = tok_t.shape
    grid = (V + _TBLK - 1) // _TBLK

    def body(in_ref, eye_ref, out_ref):
        out_ref[:, :D] = jax.lax.dot_general(
            in_ref[...],
            eye_ref[...],
            (((0,), (0,)), ((), ())),
            preferred_element_type=jnp.float32,
        )

    return pl.pallas_call(
        body,
        grid=(grid,),
        in_specs=[
            pl.BlockSpec((D, _TBLK), lambda j: (0, j)),
            pl.BlockSpec((D, D), lambda j: (0, 0)),
        ],
        out_specs=pl.BlockSpec((_TBLK, _DP), lambda j: (j, 0)),
        out_shape=jax.ShapeDtypeStruct((V, _DP), jnp.float32),
    )(tok_t, eye)


def _pe_table(max_len, d_embed):
    pos = jnp.arange(max_len, dtype=jnp.float32)[:, None]
    i = jnp.arange(0, d_embed, 2, dtype=jnp.float32)[None, :]
    angle = pos / jnp.power(10000.0, i / d_embed)
    pe = jnp.zeros((max_len, d_embed), dtype=jnp.float32)
    pe = pe.at[:, 0::2].set(jnp.sin(angle))
    pe = pe.at[:, 1::2].set(jnp.cos(angle))
    return pe


def _prep_table(tok_t, tok_tail):
    D, V = tok_t.shape
    n_full = V // 128
    tail = V - n_full * 128
    n_blocks = n_full + (1 if tail else 0)
    per_w = (n_blocks + _NW - 1) // _NW
    vp2 = (n_full + 1) * 64
    mesh = plsc.VectorSubcoreMesh(core_axis_name="c", subcore_axis_name="s")

    _PB = 4

    @functools.partial(
        pl.kernel,
        out_type=jax.ShapeDtypeStruct((vp2, 2 * D), jnp.float32),
        mesh=mesh,
        scratch_types=[
            [pltpu.VMEM((D, 128), jnp.float32)] * _PB,
            [pltpu.VMEM((64, 2 * D), jnp.float32)] * _PB,
            [pltpu.SemaphoreType.DMA] * _PB,
            [pltpu.SemaphoreType.DMA] * _PB,
        ],
        compiler_params=pltpu.CompilerParams(needs_layout_passes=False),
    )
    def prep(tt_hbm, ttail_hbm, out_hbm, ibufs, obufs, isems, osems):
        wid = lax.axis_index("s") * _NC + lax.axis_index("c")

        def cid(g):
            return wid * per_w + g

        def valid(g):
            return (g >= 0) & (g < per_w) & (cid(g) < n_blocks)

        def fire_load(g, bi):
            @pl.when(valid(g))
            def _():
                c = cid(g)

                @pl.when(c < n_full)
                def _():
                    pltpu.async_copy(
                        tt_hbm.at[:, pl.ds(c * 128, 128)], ibufs[bi], isems[bi]
                    )

                @pl.when(c == n_full)
                def _():
                    pltpu.async_copy(ttail_hbm, ibufs[bi], isems[bi])

        def wait_load(bi):
            pltpu.make_async_copy(
                tt_hbm.at[:, pl.ds(0, 128)], ibufs[bi], isems[bi]
            ).wait()

        def wait_store(bi):
            pltpu.make_async_copy(
                obufs[bi], out_hbm.at[pl.ds(0, 64)], osems[bi]
            ).wait()

        dvs = [lax.iota(jnp.int32, 16) + (16 * k) % D for k in range(2 * D // 16)]

        def transpose_block(bi):
            ib = ibufs[bi]
            ob = obufs[bi]

            @plsc.parallel_loop(0, 64, 1, unroll=4)
            def rows(j):
                for h in range(2):
                    iv = jnp.zeros((16,), jnp.int32) + (2 * j + h)
                    for k4 in range(D // 16):
                        k = h * (D // 16) + k4
                        ob[j, pl.ds(16 * k, 16)] = plsc.load_gather(
                            ib, [dvs[k], iv]
                        )

        fire_load(0, 0)
        fire_load(1, 1)

        def step(g, bi):
            @pl.when(valid(g - _PB))
            def _():
                wait_store(bi)

            @pl.when(valid(g))
            def _():
                wait_load(bi)
                transpose_block(bi)
                pltpu.async_copy(
                    obufs[bi], out_hbm.at[pl.ds(cid(g) * 64, 64)], osems[bi]
                )

            fire_load(g + 2, (bi + 2) % _PB)

        def group(t, _):
            for b in range(_PB):
                step(t * _PB + b, b)
            return 0

        n_groups = (per_w + _PB - 1) // _PB
        lax.fori_loop(0, n_groups, group, 0)
        for g_e in range(n_groups * _PB - _PB, n_groups * _PB):
            @pl.when(valid(g_e))
            def _(g_e=g_e):
                wait_store(g_e % _PB)

    return prep(tok_t, tok_tail)


@functools.partial(jax.jit, static_argnums=(3, 4, 5))
def _emb_call(tok_lin, idx, pe_ext, N, D, S):
    n_per_w = N // _NW
    n_chunks = n_per_w // _CH
    assert n_chunks % _NBUF == 0
    pe_rows = pe_ext.shape[0]
    mesh = plsc.VectorSubcoreMesh(core_axis_name="c", subcore_axis_name="s")

    @functools.partial(
        pl.kernel,
        out_type=jax.ShapeDtypeStruct((N, _DP), jnp.float32),
        mesh=mesh,
        scratch_types=[
            pltpu.VMEM((n_per_w,), jnp.int32),
            [pltpu.VMEM((_CH, D), jnp.float32)] * _NBUF,
            pltpu.VMEM((pe_rows, D), jnp.float32),
            [pltpu.SemaphoreType.DMA] * _NBUF,
            [pltpu.SemaphoreType.DMA] * _NBUF,
        ],
        compiler_params=pltpu.CompilerParams(use_tc_tiling_on_sc=False),
    )
    def emb(tok_hbm, idx_hbm, pe_hbm, out_hbm, idx_v, bufs, pe_v, gsems, ssems):
        wid = lax.axis_index("s") * _NC + lax.axis_index("c")
        base = wid * n_per_w
        pltpu.sync_copy(idx_hbm.at[pl.ds(base, n_per_w)], idx_v)
        pltpu.sync_copy(pe_hbm, pe_v)

        def fire_gather(g, bi):
            pltpu.async_copy(
                tok_hbm.at[idx_v.at[pl.ds(g * _CH, _CH)]], bufs[bi], gsems[bi]
            )

        def wait_gather(bi):
            pltpu.make_async_copy(
                tok_hbm.at[idx_v.at[pl.ds(0, _CH)]], bufs[bi], gsems[bi]
            ).wait()

        def wait_store(bi):
            pltpu.make_async_copy(
                bufs[bi], out_hbm.at[pl.ds(0, _CH), pl.ds(0, D)], ssems[bi]
            ).wait()

        fire_gather(0, 0)
        fire_gather(1, 1)

        def process(g, bi):
            wait_gather(bi)
            start = lax.rem(base + g * _CH, S)
            buf = bufs[bi]

            @plsc.parallel_loop(0, _CH, 1, unroll=8)
            def radd(r):
                p = start + r
                for k in range(D // 16):
                    sl = pl.ds(k * 16, 16)
                    buf[r, sl] = buf[r, sl] + pe_v[p, sl]

            pltpu.async_copy(
                buf,
                out_hbm.at[pl.ds(base + g * _CH, _CH), pl.ds(0, D)],
                ssems[bi],
            )

            nb = (bi + 2) % _NBUF

            @pl.when(g >= 2)
            def _():
                wait_store(nb)

            @pl.when(g + 2 < n_chunks)
            def _():
                fire_gather(g + 2, nb)

        def group(t, _):
            for b in range(_NBUF):
                process(t * _NBUF + b, b)
            return 0

        lax.fori_loop(0, n_chunks // _NBUF, group, 0)
        wait_store(_NBUF - 2)
        wait_store(_NBUF - 1)

    return emb(tok_lin, idx, pe_ext)


def kernel(x, tok_table):
    B, S = x.shape
    V, D = tok_table.shape
    N = B * S
    idx = x.reshape(N).astype(jnp.int32) * 2
    pe = _pe_table(S, D)
    pe_ext = jnp.concatenate([pe, pe[:_CH]], axis=0)
    tok_pad = _prep_table_tc(tok_table.T, jnp.eye(D, dtype=jnp.float32))
    tok_lin = tok_pad.reshape(V * 2, D)
    out = _emb_call(tok_lin, idx, pe_ext, N, D, S)
    return out.reshape(B, S, _DP)[:, :, :D]


# speedup vs baseline: 1.3299x; 1.0019x over previous
import functools

import jax
import jax.numpy as jnp
from jax import lax
from jax.experimental import pallas as pl
from jax.experimental.pallas import tpu as pltpu
from jax.experimental.pallas import tpu_sc as plsc

_NC = 2
_NS = 16
_NW = _NC * _NS
_CH = 128
_NBUF = 4
_DP = 128


_TBLK = 1024


def _prep_table_tc(tok_t, eye):
    D, V = tok_t.shape
    grid = (V + _TBLK - 1) // _TBLK

    def body(in_ref, eye_ref, out_ref):
        out_ref[:, :D] = jax.lax.dot_general(
            in_ref[...],
            eye_ref[...],
            (((0,), (0,)), ((), ())),
            preferred_element_type=jnp.float32,
        )

    return pl.pallas_call(
        body,
        grid=(grid,),
        in_specs=[
            pl.BlockSpec((D, _TBLK), lambda j: (0, j)),
            pl.BlockSpec((D, D), lambda j: (0, 0)),
        ],
        out_specs=pl.BlockSpec((_TBLK, _DP), lambda j: (j, 0)),
        out_shape=jax.ShapeDtypeStruct((V, _DP), jnp.float32),
        compiler_params=pltpu.CompilerParams(fuse_transposed_lhs_in_matmul=True),
    )(tok_t, eye)


def _pe_table(max_len, d_embed):
    pos = jnp.arange(max_len, dtype=jnp.float32)[:, None]
    i = jnp.arange(0, d_embed, 2, dtype=jnp.float32)[None, :]
    angle = pos / jnp.power(10000.0, i / d_embed)
    pe = jnp.zeros((max_len, d_embed), dtype=jnp.float32)
    pe = pe.at[:, 0::2].set(jnp.sin(angle))
    pe = pe.at[:, 1::2].set(jnp.cos(angle))
    return pe


def _prep_table(tok_t, tok_tail):
    D, V = tok_t.shape
    n_full = V // 128
    tail = V - n_full * 128
    n_blocks = n_full + (1 if tail else 0)
    per_w = (n_blocks + _NW - 1) // _NW
    vp2 = (n_full + 1) * 64
    mesh = plsc.VectorSubcoreMesh(core_axis_name="c", subcore_axis_name="s")

    _PB = 4

    @functools.partial(
        pl.kernel,
        out_type=jax.ShapeDtypeStruct((vp2, 2 * D), jnp.float32),
        mesh=mesh,
        scratch_types=[
            [pltpu.VMEM((D, 128), jnp.float32)] * _PB,
            [pltpu.VMEM((64, 2 * D), jnp.float32)] * _PB,
            [pltpu.SemaphoreType.DMA] * _PB,
            [pltpu.SemaphoreType.DMA] * _PB,
        ],
        compiler_params=pltpu.CompilerParams(needs_layout_passes=False),
    )
    def prep(tt_hbm, ttail_hbm, out_hbm, ibufs, obufs, isems, osems):
        wid = lax.axis_index("s") * _NC + lax.axis_index("c")

        def cid(g):
            return wid * per_w + g

        def valid(g):
            return (g >= 0) & (g < per_w) & (cid(g) < n_blocks)

        def fire_load(g, bi):
            @pl.when(valid(g))
            def _():
                c = cid(g)

                @pl.when(c < n_full)
                def _():
                    pltpu.async_copy(
                        tt_hbm.at[:, pl.ds(c * 128, 128)], ibufs[bi], isems[bi]
                    )

                @pl.when(c == n_full)
                def _():
                    pltpu.async_copy(ttail_hbm, ibufs[bi], isems[bi])

        def wait_load(bi):
            pltpu.make_async_copy(
                tt_hbm.at[:, pl.ds(0, 128)], ibufs[bi], isems[bi]
            ).wait()

        def wait_store(bi):
            pltpu.make_async_copy(
                obufs[bi], out_hbm.at[pl.ds(0, 64)], osems[bi]
            ).wait()

        dvs = [lax.iota(jnp.int32, 16) + (16 * k) % D for k in range(2 * D // 16)]

        def transpose_block(bi):
            ib = ibufs[bi]
            ob = obufs[bi]

            @plsc.parallel_loop(0, 64, 1, unroll=4)
            def rows(j):
                for h in range(2):
                    iv = jnp.zeros((16,), jnp.int32) + (2 * j + h)
                    for k4 in range(D // 16):
                        k = h * (D // 16) + k4
                        ob[j, pl.ds(16 * k, 16)] = plsc.load_gather(
                            ib, [dvs[k], iv]
                        )

        fire_load(0, 0)
        fire_load(1, 1)

        def step(g, bi):
            @pl.when(valid(g - _PB))
            def _():
                wait_store(bi)

            @pl.when(valid(g))
            def _():
                wait_load(bi)
                transpose_block(bi)
                pltpu.async_copy(
                    obufs[bi], out_hbm.at[pl.ds(cid(g) * 64, 64)], osems[bi]
                )

            fire_load(g + 2, (bi + 2) % _PB)

        def group(t, _):
            for b in range(_PB):
                step(t * _PB + b, b)
            return 0

        n_groups = (per_w + _PB - 1) // _PB
        lax.fori_loop(0, n_groups, group, 0)
        for g_e in range(n_groups * _PB - _PB, n_groups * _PB):
            @pl.when(valid(g_e))
            def _(g_e=g_e):
                wait_store(g_e % _PB)

    return prep(tok_t, tok_tail)


@functools.partial(jax.jit, static_argnums=(3, 4, 5))
def _emb_call(tok_lin, idx, pe_ext, N, D, S):
    n_per_w = N // _NW
    n_chunks = n_per_w // _CH
    assert n_chunks % _NBUF == 0
    pe_rows = pe_ext.shape[0]
    mesh = plsc.VectorSubcoreMesh(core_axis_name="c", subcore_axis_name="s")

    @functools.partial(
        pl.kernel,
        out_type=jax.ShapeDtypeStruct((N, _DP), jnp.float32),
        mesh=mesh,
        scratch_types=[
            pltpu.VMEM((n_per_w,), jnp.int32),
            [pltpu.VMEM((_CH, D), jnp.float32)] * _NBUF,
            pltpu.VMEM((pe_rows, D), jnp.float32),
            [pltpu.SemaphoreType.DMA] * _NBUF,
            [pltpu.SemaphoreType.DMA] * _NBUF,
        ],
        compiler_params=pltpu.CompilerParams(use_tc_tiling_on_sc=False),
    )
    def emb(tok_hbm, idx_hbm, pe_hbm, out_hbm, idx_v, bufs, pe_v, gsems, ssems):
        wid = lax.axis_index("s") * _NC + lax.axis_index("c")
        base = wid * n_per_w
        pltpu.sync_copy(idx_hbm.at[pl.ds(base, n_per_w)], idx_v)
        pltpu.sync_copy(pe_hbm, pe_v)

        def fire_gather(g, bi):
            pltpu.async_copy(
                tok_hbm.at[idx_v.at[pl.ds(g * _CH, _CH)]], bufs[bi], gsems[bi]
            )

        def wait_gather(bi):
            pltpu.make_async_copy(
                tok_hbm.at[idx_v.at[pl.ds(0, _CH)]], bufs[bi], gsems[bi]
            ).wait()

        def wait_store(bi):
            pltpu.make_async_copy(
                bufs[bi], out_hbm.at[pl.ds(0, _CH), pl.ds(0, D)], ssems[bi]
            ).wait()

        fire_gather(0, 0)
        fire_gather(1, 1)

        def process(g, bi):
            wait_gather(bi)
            start = lax.rem(base + g * _CH, S)
            buf = bufs[bi]

            @plsc.parallel_loop(0, _CH, 1, unroll=8)
            def radd(r):
                p = start + r
                for k in range(D // 16):
                    sl = pl.ds(k * 16, 16)
                    buf[r, sl] = buf[r, sl] + pe_v[p, sl]

            pltpu.async_copy(
                buf,
                out_hbm.at[pl.ds(base + g * _CH, _CH), pl.ds(0, D)],
                ssems[bi],
            )

            nb = (bi + 2) % _NBUF

            @pl.when(g >= 2)
            def _():
                wait_store(nb)

            @pl.when(g + 2 < n_chunks)
            def _():
                fire_gather(g + 2, nb)

        def group(t, _):
            for b in range(_NBUF):
                process(t * _NBUF + b, b)
            return 0

        lax.fori_loop(0, n_chunks // _NBUF, group, 0)
        wait_store(_NBUF - 2)
        wait_store(_NBUF - 1)

    return emb(tok_lin, idx, pe_ext)


def kernel(x, tok_table):
    B, S = x.shape
    V, D = tok_table.shape
    N = B * S
    idx = x.reshape(N).astype(jnp.int32) * 2
    pe = _pe_table(S, D)
    pe_ext = jnp.concatenate([pe, pe[:_CH]], axis=0)
    tok_pad = _prep_table_tc(tok_table.T, jnp.eye(D, dtype=jnp.float32))
    tok_lin = tok_pad.reshape(V * 2, D)
    out = _emb_call(tok_lin, idx, pe_ext, N, D, S)
    return out.reshape(B, S, _DP)[:, :, :D]


# speedup vs baseline: 2.0561x; 1.5461x over previous
import functools

import jax
import jax.numpy as jnp
from jax import lax
from jax.experimental import pallas as pl
from jax.experimental.pallas import tpu as pltpu
from jax.experimental.pallas import tpu_sc as plsc

_NC = 2
_NS = 16
_NW = _NC * _NS
_CH = 128
_NBUF = 4
_DP = 128


_TBLK = 4096


def _prep_table_tc(tok_t, eye):
    D, V = tok_t.shape
    grid = (V + _TBLK - 1) // _TBLK

    def body(in_ref, eye_ref, out_ref):
        out_ref[:, :D] = jax.lax.dot_general(
            in_ref[...],
            eye_ref[...],
            (((0,), (0,)), ((), ())),
            preferred_element_type=jnp.float32,
        )

    return pl.pallas_call(
        body,
        grid=(grid,),
        in_specs=[
            pl.BlockSpec((D, _TBLK), lambda j: (0, j)),
            pl.BlockSpec((D, D), lambda j: (0, 0)),
        ],
        out_specs=pl.BlockSpec((_TBLK, _DP), lambda j: (j, 0)),
        out_shape=jax.ShapeDtypeStruct((V, _DP), jnp.float32),
        compiler_params=pltpu.CompilerParams(fuse_transposed_lhs_in_matmul=True),
    )(tok_t, eye)


def _pe_table(max_len, d_embed):
    pos = jnp.arange(max_len, dtype=jnp.float32)[:, None]
    i = jnp.arange(0, d_embed, 2, dtype=jnp.float32)[None, :]
    angle = pos / jnp.power(10000.0, i / d_embed)
    pe = jnp.zeros((max_len, d_embed), dtype=jnp.float32)
    pe = pe.at[:, 0::2].set(jnp.sin(angle))
    pe = pe.at[:, 1::2].set(jnp.cos(angle))
    return pe


def _prep_table(tok_t, tok_tail):
    D, V = tok_t.shape
    n_full = V // 128
    tail = V - n_full * 128
    n_blocks = n_full + (1 if tail else 0)
    per_w = (n_blocks + _NW - 1) // _NW
    vp2 = (n_full + 1) * 64
    mesh = plsc.VectorSubcoreMesh(core_axis_name="c", subcore_axis_name="s")

    _PB = 4

    @functools.partial(
        pl.kernel,
        out_type=jax.ShapeDtypeStruct((vp2, 2 * D), jnp.float32),
        mesh=mesh,
        scratch_types=[
            [pltpu.VMEM((D, 128), jnp.float32)] * _PB,
            [pltpu.VMEM((64, 2 * D), jnp.float32)] * _PB,
            [pltpu.SemaphoreType.DMA] * _PB,
            [pltpu.SemaphoreType.DMA] * _PB,
        ],
        compiler_params=pltpu.CompilerParams(needs_layout_passes=False),
    )
    def prep(tt_hbm, ttail_hbm, out_hbm, ibufs, obufs, isems, osems):
        wid = lax.axis_index("s") * _NC + lax.axis_index("c")

        def cid(g):
            return wid * per_w + g

        def valid(g):
            return (g >= 0) & (g < per_w) & (cid(g) < n_blocks)

        def fire_load(g, bi):
            @pl.when(valid(g))
            def _():
                c = cid(g)

                @pl.when(c < n_full)
                def _():
                    pltpu.async_copy(
                        tt_hbm.at[:, pl.ds(c * 128, 128)], ibufs[bi], isems[bi]
                    )

                @pl.when(c == n_full)
                def _():
                    pltpu.async_copy(ttail_hbm, ibufs[bi], isems[bi])

        def wait_load(bi):
            pltpu.make_async_copy(
                tt_hbm.at[:, pl.ds(0, 128)], ibufs[bi], isems[bi]
            ).wait()

        def wait_store(bi):
            pltpu.make_async_copy(
                obufs[bi], out_hbm.at[pl.ds(0, 64)], osems[bi]
            ).wait()

        dvs = [lax.iota(jnp.int32, 16) + (16 * k) % D for k in range(2 * D // 16)]

        def transpose_block(bi):
            ib = ibufs[bi]
            ob = obufs[bi]

            @plsc.parallel_loop(0, 64, 1, unroll=4)
            def rows(j):
                for h in range(2):
                    iv = jnp.zeros((16,), jnp.int32) + (2 * j + h)
                    for k4 in range(D // 16):
                        k = h * (D // 16) + k4
                        ob[j, pl.ds(16 * k, 16)] = plsc.load_gather(
                            ib, [dvs[k], iv]
                        )

        fire_load(0, 0)
        fire_load(1, 1)

        def step(g, bi):
            @pl.when(valid(g - _PB))
            def _():
                wait_store(bi)

            @pl.when(valid(g))
            def _():
                wait_load(bi)
                transpose_block(bi)
                pltpu.async_copy(
                    obufs[bi], out_hbm.at[pl.ds(cid(g) * 64, 64)], osems[bi]
                )

            fire_load(g + 2, (bi + 2) % _PB)

        def group(t, _):
            for b in range(_PB):
                step(t * _PB + b, b)
            return 0

        n_groups = (per_w + _PB - 1) // _PB
        lax.fori_loop(0, n_groups, group, 0)
        for g_e in range(n_groups * _PB - _PB, n_groups * _PB):
            @pl.when(valid(g_e))
            def _(g_e=g_e):
                wait_store(g_e % _PB)

    return prep(tok_t, tok_tail)


@functools.partial(jax.jit, static_argnums=(3, 4, 5))
def _emb_call(tok_lin, idx, pe_ext, N, D, S):
    n_per_w = N // _NW
    n_chunks = n_per_w // _CH
    assert n_chunks % _NBUF == 0
    pe_rows = pe_ext.shape[0]
    mesh = plsc.VectorSubcoreMesh(core_axis_name="c", subcore_axis_name="s")

    @functools.partial(
        pl.kernel,
        out_type=jax.ShapeDtypeStruct((N, _DP), jnp.float32),
        mesh=mesh,
        scratch_types=[
            pltpu.VMEM((n_per_w,), jnp.int32),
            [pltpu.VMEM((_CH, D), jnp.float32)] * _NBUF,
            pltpu.VMEM((pe_rows, D), jnp.float32),
            [pltpu.SemaphoreType.DMA] * _NBUF,
            [pltpu.SemaphoreType.DMA] * _NBUF,
        ],
        compiler_params=pltpu.CompilerParams(use_tc_tiling_on_sc=False),
    )
    def emb(tok_hbm, idx_hbm, pe_hbm, out_hbm, idx_v, bufs, pe_v, gsems, ssems):
        wid = lax.axis_index("s") * _NC + lax.axis_index("c")
        base = wid * n_per_w
        pltpu.sync_copy(idx_hbm.at[pl.ds(base, n_per_w)], idx_v)
        pltpu.sync_copy(pe_hbm, pe_v)

        def fire_gather(g, bi):
            pltpu.async_copy(
                tok_hbm.at[idx_v.at[pl.ds(g * _CH, _CH)]], bufs[bi], gsems[bi]
            )

        def wait_gather(bi):
            pltpu.make_async_copy(
                tok_hbm.at[idx_v.at[pl.ds(0, _CH)]], bufs[bi], gsems[bi]
            ).wait()

        def wait_store(bi):
            pltpu.make_async_copy(
                bufs[bi], out_hbm.at[pl.ds(0, _CH), pl.ds(0, D)], ssems[bi]
            ).wait()

        fire_gather(0, 0)
        fire_gather(1, 1)

        def process(g, bi):
            wait_gather(bi)
            start = lax.rem(base + g * _CH, S)
            buf = bufs[bi]

            @plsc.parallel_loop(0, _CH, 1, unroll=8)
            def radd(r):
                p = start + r
                for k in range(D // 16):
                    sl = pl.ds(k * 16, 16)
                    buf[r, sl] = buf[r, sl] + pe_v[p, sl]

            pltpu.async_copy(
                buf,
                out_hbm.at[pl.ds(base + g * _CH, _CH), pl.ds(0, D)],
                ssems[bi],
            )

            nb = (bi + 2) % _NBUF

            @pl.when(g >= 2)
            def _():
                wait_store(nb)

            @pl.when(g + 2 < n_chunks)
            def _():
                fire_gather(g + 2, nb)

        def group(t, _):
            for b in range(_NBUF):
                process(t * _NBUF + b, b)
            return 0

        lax.fori_loop(0, n_chunks // _NBUF, group, 0)
        wait_store(_NBUF - 2)
        wait_store(_NBUF - 1)

    return emb(tok_lin, idx, pe_ext)


def kernel(x, tok_table):
    B, S = x.shape
    V, D = tok_table.shape
    N = B * S
    idx = x.reshape(N).astype(jnp.int32) * 2
    pe = _pe_table(S, D)
    pe_ext = jnp.concatenate([pe, pe[:_CH]], axis=0)
    tok_pad = _prep_table_tc(tok_table.T, jnp.eye(D, dtype=jnp.float32))
    tok_lin = tok_pad.reshape(V * 2, D)
    out = _emb_call(tok_lin, idx, pe_ext, N, D, S)
    return out.reshape(B, S, _DP)[:, :, :D]


# speedup vs baseline: 2.3005x; 1.1189x over previous
import functools

import jax
import jax.numpy as jnp
from jax import lax
from jax.experimental import pallas as pl
from jax.experimental.pallas import tpu as pltpu
from jax.experimental.pallas import tpu_sc as plsc

_NC = 2
_NS = 16
_NW = _NC * _NS
_CH = 128
_NBUF = 4
_DP = 128


_TBLK = 8192


def _prep_table_tc(tok_t, eye):
    D, V = tok_t.shape
    grid = (V + _TBLK - 1) // _TBLK

    def body(in_ref, eye_ref, out_ref):
        out_ref[:, :D] = jax.lax.dot_general(
            in_ref[...],
            eye_ref[...],
            (((0,), (0,)), ((), ())),
            preferred_element_type=jnp.float32,
        )

    return pl.pallas_call(
        body,
        grid=(grid,),
        in_specs=[
            pl.BlockSpec((D, _TBLK), lambda j: (0, j)),
            pl.BlockSpec((D, D), lambda j: (0, 0)),
        ],
        out_specs=pl.BlockSpec((_TBLK, _DP), lambda j: (j, 0)),
        out_shape=jax.ShapeDtypeStruct((V, _DP), jnp.float32),
        compiler_params=pltpu.CompilerParams(fuse_transposed_lhs_in_matmul=True),
    )(tok_t, eye)


def _pe_table(max_len, d_embed):
    pos = jnp.arange(max_len, dtype=jnp.float32)[:, None]
    i = jnp.arange(0, d_embed, 2, dtype=jnp.float32)[None, :]
    angle = pos / jnp.power(10000.0, i / d_embed)
    pe = jnp.zeros((max_len, d_embed), dtype=jnp.float32)
    pe = pe.at[:, 0::2].set(jnp.sin(angle))
    pe = pe.at[:, 1::2].set(jnp.cos(angle))
    return pe


def _prep_table(tok_t, tok_tail):
    D, V = tok_t.shape
    n_full = V // 128
    tail = V - n_full * 128
    n_blocks = n_full + (1 if tail else 0)
    per_w = (n_blocks + _NW - 1) // _NW
    vp2 = (n_full + 1) * 64
    mesh = plsc.VectorSubcoreMesh(core_axis_name="c", subcore_axis_name="s")

    _PB = 4

    @functools.partial(
        pl.kernel,
        out_type=jax.ShapeDtypeStruct((vp2, 2 * D), jnp.float32),
        mesh=mesh,
        scratch_types=[
            [pltpu.VMEM((D, 128), jnp.float32)] * _PB,
            [pltpu.VMEM((64, 2 * D), jnp.float32)] * _PB,
            [pltpu.SemaphoreType.DMA] * _PB,
            [pltpu.SemaphoreType.DMA] * _PB,
        ],
        compiler_params=pltpu.CompilerParams(needs_layout_passes=False),
    )
    def prep(tt_hbm, ttail_hbm, out_hbm, ibufs, obufs, isems, osems):
        wid = lax.axis_index("s") * _NC + lax.axis_index("c")

        def cid(g):
            return wid * per_w + g

        def valid(g):
            return (g >= 0) & (g < per_w) & (cid(g) < n_blocks)

        def fire_load(g, bi):
            @pl.when(valid(g))
            def _():
                c = cid(g)

                @pl.when(c < n_full)
                def _():
                    pltpu.async_copy(
                        tt_hbm.at[:, pl.ds(c * 128, 128)], ibufs[bi], isems[bi]
                    )

                @pl.when(c == n_full)
                def _():
                    pltpu.async_copy(ttail_hbm, ibufs[bi], isems[bi])

        def wait_load(bi):
            pltpu.make_async_copy(
                tt_hbm.at[:, pl.ds(0, 128)], ibufs[bi], isems[bi]
            ).wait()

        def wait_store(bi):
            pltpu.make_async_copy(
                obufs[bi], out_hbm.at[pl.ds(0, 64)], osems[bi]
            ).wait()

        dvs = [lax.iota(jnp.int32, 16) + (16 * k) % D for k in range(2 * D // 16)]

        def transpose_block(bi):
            ib = ibufs[bi]
            ob = obufs[bi]

            @plsc.parallel_loop(0, 64, 1, unroll=4)
            def rows(j):
                for h in range(2):
                    iv = jnp.zeros((16,), jnp.int32) + (2 * j + h)
                    for k4 in range(D // 16):
                        k = h * (D // 16) + k4
                        ob[j, pl.ds(16 * k, 16)] = plsc.load_gather(
                            ib, [dvs[k], iv]
                        )

        fire_load(0, 0)
        fire_load(1, 1)

        def step(g, bi):
            @pl.when(valid(g - _PB))
            def _():
                wait_store(bi)

            @pl.when(valid(g))
            def _():
                wait_load(bi)
                transpose_block(bi)
                pltpu.async_copy(
                    obufs[bi], out_hbm.at[pl.ds(cid(g) * 64, 64)], osems[bi]
                )

            fire_load(g + 2, (bi + 2) % _PB)

        def group(t, _):
            for b in range(_PB):
                step(t * _PB + b, b)
            return 0

        n_groups = (per_w + _PB - 1) // _PB
        lax.fori_loop(0, n_groups, group, 0)
        for g_e in range(n_groups * _PB - _PB, n_groups * _PB):
            @pl.when(valid(g_e))
            def _(g_e=g_e):
                wait_store(g_e % _PB)

    return prep(tok_t, tok_tail)


@functools.partial(jax.jit, static_argnums=(3, 4, 5))
def _emb_call(tok_lin, idx, pe_ext, N, D, S):
    n_per_w = N // _NW
    n_chunks = n_per_w // _CH
    assert n_chunks % _NBUF == 0
    pe_rows = pe_ext.shape[0]
    mesh = plsc.VectorSubcoreMesh(core_axis_name="c", subcore_axis_name="s")

    @functools.partial(
        pl.kernel,
        out_type=jax.ShapeDtypeStruct((N, _DP), jnp.float32),
        mesh=mesh,
        scratch_types=[
            pltpu.VMEM((n_per_w,), jnp.int32),
            [pltpu.VMEM((_CH, D), jnp.float32)] * _NBUF,
            pltpu.VMEM((pe_rows, D), jnp.float32),
            [pltpu.SemaphoreType.DMA] * _NBUF,
            [pltpu.SemaphoreType.DMA] * _NBUF,
        ],
        compiler_params=pltpu.CompilerParams(use_tc_tiling_on_sc=False),
    )
    def emb(tok_hbm, idx_hbm, pe_hbm, out_hbm, idx_v, bufs, pe_v, gsems, ssems):
        wid = lax.axis_index("s") * _NC + lax.axis_index("c")
        base = wid * n_per_w
        pltpu.sync_copy(idx_hbm.at[pl.ds(base, n_per_w)], idx_v)
        pltpu.sync_copy(pe_hbm, pe_v)

        def fire_gather(g, bi):
            pltpu.async_copy(
                tok_hbm.at[idx_v.at[pl.ds(g * _CH, _CH)]], bufs[bi], gsems[bi]
            )

        def wait_gather(bi):
            pltpu.make_async_copy(
                tok_hbm.at[idx_v.at[pl.ds(0, _CH)]], bufs[bi], gsems[bi]
            ).wait()

        def wait_store(bi):
            pltpu.make_async_copy(
                bufs[bi], out_hbm.at[pl.ds(0, _CH), pl.ds(0, D)], ssems[bi]
            ).wait()

        fire_gather(0, 0)
        fire_gather(1, 1)

        def process(g, bi):
            wait_gather(bi)
            start = lax.rem(base + g * _CH, S)
            buf = bufs[bi]

            @plsc.parallel_loop(0, _CH, 1, unroll=8)
            def radd(r):
                p = start + r
                for k in range(D // 16):
                    sl = pl.ds(k * 16, 16)
                    buf[r, sl] = buf[r, sl] + pe_v[p, sl]

            pltpu.async_copy(
                buf,
                out_hbm.at[pl.ds(base + g * _CH, _CH), pl.ds(0, D)],
                ssems[bi],
            )

            nb = (bi + 2) % _NBUF

            @pl.when(g >= 2)
            def _():
                wait_store(nb)

            @pl.when(g + 2 < n_chunks)
            def _():
                fire_gather(g + 2, nb)

        def group(t, _):
            for b in range(_NBUF):
                process(t * _NBUF + b, b)
            return 0

        lax.fori_loop(0, n_chunks // _NBUF, group, 0)
        wait_store(_NBUF - 2)
        wait_store(_NBUF - 1)

    return emb(tok_lin, idx, pe_ext)


def kernel(x, tok_table):
    B, S = x.shape
    V, D = tok_table.shape
    N = B * S
    idx = x.reshape(N).astype(jnp.int32) * 2
    pe = _pe_table(S, D)
    pe_ext = jnp.concatenate([pe, pe[:_CH]], axis=0)
    tok_pad = _prep_table_tc(tok_table.T, jnp.eye(D, dtype=jnp.float32))
    tok_lin = tok_pad.reshape(V * 2, D)
    out = _emb_call(tok_lin, idx, pe_ext, N, D, S)
    return out.reshape(B, S, _DP)[:, :, :D]


# speedup vs baseline: 2.3843x; 1.0364x over previous
import functools

import jax
import jax.numpy as jnp
from jax import lax
from jax.experimental import pallas as pl
from jax.experimental.pallas import tpu as pltpu
from jax.experimental.pallas import tpu_sc as plsc

_NC = 2
_NS = 16
_NW = _NC * _NS
_CH = 128
_NBUF = 4
_DP = 128


_TBLK = 16384


def _prep_table_tc(tok_t, eye):
    D, V = tok_t.shape
    grid = (V + _TBLK - 1) // _TBLK

    def body(in_ref, eye_ref, out_ref):
        out_ref[:, :D] = jax.lax.dot_general(
            in_ref[...],
            eye_ref[...],
            (((0,), (0,)), ((), ())),
            preferred_element_type=jnp.float32,
        )

    return pl.pallas_call(
        body,
        grid=(grid,),
        in_specs=[
            pl.BlockSpec((D, _TBLK), lambda j: (0, j)),
            pl.BlockSpec((D, D), lambda j: (0, 0)),
        ],
        out_specs=pl.BlockSpec((_TBLK, _DP), lambda j: (j, 0)),
        out_shape=jax.ShapeDtypeStruct((V, _DP), jnp.float32),
        compiler_params=pltpu.CompilerParams(fuse_transposed_lhs_in_matmul=True),
    )(tok_t, eye)


def _pe_table(max_len, d_embed):
    pos = jnp.arange(max_len, dtype=jnp.float32)[:, None]
    i = jnp.arange(0, d_embed, 2, dtype=jnp.float32)[None, :]
    angle = pos / jnp.power(10000.0, i / d_embed)
    pe = jnp.zeros((max_len, d_embed), dtype=jnp.float32)
    pe = pe.at[:, 0::2].set(jnp.sin(angle))
    pe = pe.at[:, 1::2].set(jnp.cos(angle))
    return pe


def _prep_table(tok_t, tok_tail):
    D, V = tok_t.shape
    n_full = V // 128
    tail = V - n_full * 128
    n_blocks = n_full + (1 if tail else 0)
    per_w = (n_blocks + _NW - 1) // _NW
    vp2 = (n_full + 1) * 64
    mesh = plsc.VectorSubcoreMesh(core_axis_name="c", subcore_axis_name="s")

    _PB = 4

    @functools.partial(
        pl.kernel,
        out_type=jax.ShapeDtypeStruct((vp2, 2 * D), jnp.float32),
        mesh=mesh,
        scratch_types=[
            [pltpu.VMEM((D, 128), jnp.float32)] * _PB,
            [pltpu.VMEM((64, 2 * D), jnp.float32)] * _PB,
            [pltpu.SemaphoreType.DMA] * _PB,
            [pltpu.SemaphoreType.DMA] * _PB,
        ],
        compiler_params=pltpu.CompilerParams(needs_layout_passes=False),
    )
    def prep(tt_hbm, ttail_hbm, out_hbm, ibufs, obufs, isems, osems):
        wid = lax.axis_index("s") * _NC + lax.axis_index("c")

        def cid(g):
            return wid * per_w + g

        def valid(g):
            return (g >= 0) & (g < per_w) & (cid(g) < n_blocks)

        def fire_load(g, bi):
            @pl.when(valid(g))
            def _():
                c = cid(g)

                @pl.when(c < n_full)
                def _():
                    pltpu.async_copy(
                        tt_hbm.at[:, pl.ds(c * 128, 128)], ibufs[bi], isems[bi]
                    )

                @pl.when(c == n_full)
                def _():
                    pltpu.async_copy(ttail_hbm, ibufs[bi], isems[bi])

        def wait_load(bi):
            pltpu.make_async_copy(
                tt_hbm.at[:, pl.ds(0, 128)], ibufs[bi], isems[bi]
            ).wait()

        def wait_store(bi):
            pltpu.make_async_copy(
                obufs[bi], out_hbm.at[pl.ds(0, 64)], osems[bi]
            ).wait()

        dvs = [lax.iota(jnp.int32, 16) + (16 * k) % D for k in range(2 * D // 16)]

        def transpose_block(bi):
            ib = ibufs[bi]
            ob = obufs[bi]

            @plsc.parallel_loop(0, 64, 1, unroll=4)
            def rows(j):
                for h in range(2):
                    iv = jnp.zeros((16,), jnp.int32) + (2 * j + h)
                    for k4 in range(D // 16):
                        k = h * (D // 16) + k4
                        ob[j, pl.ds(16 * k, 16)] = plsc.load_gather(
                            ib, [dvs[k], iv]
                        )

        fire_load(0, 0)
        fire_load(1, 1)

        def step(g, bi):
            @pl.when(valid(g - _PB))
            def _():
                wait_store(bi)

            @pl.when(valid(g))
            def _():
                wait_load(bi)
                transpose_block(bi)
                pltpu.async_copy(
                    obufs[bi], out_hbm.at[pl.ds(cid(g) * 64, 64)], osems[bi]
                )

            fire_load(g + 2, (bi + 2) % _PB)

        def group(t, _):
            for b in range(_PB):
                step(t * _PB + b, b)
            return 0

        n_groups = (per_w + _PB - 1) // _PB
        lax.fori_loop(0, n_groups, group, 0)
        for g_e in range(n_groups * _PB - _PB, n_groups * _PB):
            @pl.when(valid(g_e))
            def _(g_e=g_e):
                wait_store(g_e % _PB)

    return prep(tok_t, tok_tail)


@functools.partial(jax.jit, static_argnums=(3, 4, 5))
def _emb_call(tok_lin, idx, pe_ext, N, D, S):
    n_per_w = N // _NW
    n_chunks = n_per_w // _CH
    assert n_chunks % _NBUF == 0
    pe_rows = pe_ext.shape[0]
    mesh = plsc.VectorSubcoreMesh(core_axis_name="c", subcore_axis_name="s")

    @functools.partial(
        pl.kernel,
        out_type=jax.ShapeDtypeStruct((N, _DP), jnp.float32),
        mesh=mesh,
        scratch_types=[
            pltpu.VMEM((n_per_w,), jnp.int32),
            [pltpu.VMEM((_CH, D), jnp.float32)] * _NBUF,
            pltpu.VMEM((pe_rows, D), jnp.float32),
            [pltpu.SemaphoreType.DMA] * _NBUF,
            [pltpu.SemaphoreType.DMA] * _NBUF,
        ],
        compiler_params=pltpu.CompilerParams(use_tc_tiling_on_sc=False),
    )
    def emb(tok_hbm, idx_hbm, pe_hbm, out_hbm, idx_v, bufs, pe_v, gsems, ssems):
        wid = lax.axis_index("s") * _NC + lax.axis_index("c")
        base = wid * n_per_w
        pltpu.sync_copy(idx_hbm.at[pl.ds(base, n_per_w)], idx_v)
        pltpu.sync_copy(pe_hbm, pe_v)

        def fire_gather(g, bi):
            pltpu.async_copy(
                tok_hbm.at[idx_v.at[pl.ds(g * _CH, _CH)]], bufs[bi], gsems[bi]
            )

        def wait_gather(bi):
            pltpu.make_async_copy(
                tok_hbm.at[idx_v.at[pl.ds(0, _CH)]], bufs[bi], gsems[bi]
            ).wait()

        def wait_store(bi):
            pltpu.make_async_copy(
                bufs[bi], out_hbm.at[pl.ds(0, _CH), pl.ds(0, D)], ssems[bi]
            ).wait()

        fire_gather(0, 0)
        fire_gather(1, 1)

        def process(g, bi):
            wait_gather(bi)
            start = lax.rem(base + g * _CH, S)
            buf = bufs[bi]

            @plsc.parallel_loop(0, _CH, 1, unroll=8)
            def radd(r):
                p = start + r
                for k in range(D // 16):
                    sl = pl.ds(k * 16, 16)
                    buf[r, sl] = buf[r, sl] + pe_v[p, sl]

            pltpu.async_copy(
                buf,
                out_hbm.at[pl.ds(base + g * _CH, _CH), pl.ds(0, D)],
                ssems[bi],
            )

            nb = (bi + 2) % _NBUF

            @pl.when(g >= 2)
            def _():
                wait_store(nb)

            @pl.when(g + 2 < n_chunks)
            def _():
                fire_gather(g + 2, nb)

        def group(t, _):
            for b in range(_NBUF):
                process(t * _NBUF + b, b)
            return 0

        lax.fori_loop(0, n_chunks // _NBUF, group, 0)
        wait_store(_NBUF - 2)
        wait_store(_NBUF - 1)

    return emb(tok_lin, idx, pe_ext)


def kernel(x, tok_table):
    B, S = x.shape
    V, D = tok_table.shape
    N = B * S
    idx = x.reshape(N).astype(jnp.int32) * 2
    pe = _pe_table(S, D)
    pe_ext = jnp.concatenate([pe, pe[:_CH]], axis=0)
    tok_pad = _prep_table_tc(tok_table.T, jnp.eye(D, dtype=jnp.float32))
    tok_lin = tok_pad.reshape(V * 2, D)
    out = _emb_call(tok_lin, idx, pe_ext, N, D, S)
    return out.reshape(B, S, _DP)[:, :, :D]


# speedup vs baseline: 2.4099x; 1.0107x over previous
import functools

import jax
import jax.numpy as jnp
from jax import lax
from jax.experimental import pallas as pl
from jax.experimental.pallas import tpu as pltpu
from jax.experimental.pallas import tpu_sc as plsc

_NC = 2
_NS = 16
_NW = _NC * _NS
_CH = 128
_NBUF = 4
_DP = 128


_TBLK = 32768


def _prep_table_tc(tok_t, eye):
    D, V = tok_t.shape
    grid = (V + _TBLK - 1) // _TBLK

    def body(in_ref, eye_ref, out_ref):
        out_ref[:, :D] = jax.lax.dot_general(
            in_ref[...],
            eye_ref[...],
            (((0,), (0,)), ((), ())),
            preferred_element_type=jnp.float32,
        )

    return pl.pallas_call(
        body,
        grid=(grid,),
        in_specs=[
            pl.BlockSpec((D, _TBLK), lambda j: (0, j)),
            pl.BlockSpec((D, D), lambda j: (0, 0)),
        ],
        out_specs=pl.BlockSpec((_TBLK, _DP), lambda j: (j, 0)),
        out_shape=jax.ShapeDtypeStruct((V, _DP), jnp.float32),
        compiler_params=pltpu.CompilerParams(fuse_transposed_lhs_in_matmul=True),
    )(tok_t, eye)


def _pe_table(max_len, d_embed):
    pos = jnp.arange(max_len, dtype=jnp.float32)[:, None]
    i = jnp.arange(0, d_embed, 2, dtype=jnp.float32)[None, :]
    angle = pos / jnp.power(10000.0, i / d_embed)
    pe = jnp.zeros((max_len, d_embed), dtype=jnp.float32)
    pe = pe.at[:, 0::2].set(jnp.sin(angle))
    pe = pe.at[:, 1::2].set(jnp.cos(angle))
    return pe


def _prep_table(tok_t, tok_tail):
    D, V = tok_t.shape
    n_full = V // 128
    tail = V - n_full * 128
    n_blocks = n_full + (1 if tail else 0)
    per_w = (n_blocks + _NW - 1) // _NW
    vp2 = (n_full + 1) * 64
    mesh = plsc.VectorSubcoreMesh(core_axis_name="c", subcore_axis_name="s")

    _PB = 4

    @functools.partial(
        pl.kernel,
        out_type=jax.ShapeDtypeStruct((vp2, 2 * D), jnp.float32),
        mesh=mesh,
        scratch_types=[
            [pltpu.VMEM((D, 128), jnp.float32)] * _PB,
            [pltpu.VMEM((64, 2 * D), jnp.float32)] * _PB,
            [pltpu.SemaphoreType.DMA] * _PB,
            [pltpu.SemaphoreType.DMA] * _PB,
        ],
        compiler_params=pltpu.CompilerParams(needs_layout_passes=False),
    )
    def prep(tt_hbm, ttail_hbm, out_hbm, ibufs, obufs, isems, osems):
        wid = lax.axis_index("s") * _NC + lax.axis_index("c")

        def cid(g):
            return wid * per_w + g

        def valid(g):
            return (g >= 0) & (g < per_w) & (cid(g) < n_blocks)

        def fire_load(g, bi):
            @pl.when(valid(g))
            def _():
                c = cid(g)

                @pl.when(c < n_full)
                def _():
                    pltpu.async_copy(
                        tt_hbm.at[:, pl.ds(c * 128, 128)], ibufs[bi], isems[bi]
                    )

                @pl.when(c == n_full)
                def _():
                    pltpu.async_copy(ttail_hbm, ibufs[bi], isems[bi])

        def wait_load(bi):
            pltpu.make_async_copy(
                tt_hbm.at[:, pl.ds(0, 128)], ibufs[bi], isems[bi]
            ).wait()

        def wait_store(bi):
            pltpu.make_async_copy(
                obufs[bi], out_hbm.at[pl.ds(0, 64)], osems[bi]
            ).wait()

        dvs = [lax.iota(jnp.int32, 16) + (16 * k) % D for k in range(2 * D // 16)]

        def transpose_block(bi):
            ib = ibufs[bi]
            ob = obufs[bi]

            @plsc.parallel_loop(0, 64, 1, unroll=4)
            def rows(j):
                for h in range(2):
                    iv = jnp.zeros((16,), jnp.int32) + (2 * j + h)
                    for k4 in range(D // 16):
                        k = h * (D // 16) + k4
                        ob[j, pl.ds(16 * k, 16)] = plsc.load_gather(
                            ib, [dvs[k], iv]
                        )

        fire_load(0, 0)
        fire_load(1, 1)

        def step(g, bi):
            @pl.when(valid(g - _PB))
            def _():
                wait_store(bi)

            @pl.when(valid(g))
            def _():
                wait_load(bi)
                transpose_block(bi)
                pltpu.async_copy(
                    obufs[bi], out_hbm.at[pl.ds(cid(g) * 64, 64)], osems[bi]
                )

            fire_load(g + 2, (bi + 2) % _PB)

        def group(t, _):
            for b in range(_PB):
                step(t * _PB + b, b)
            return 0

        n_groups = (per_w + _PB - 1) // _PB
        lax.fori_loop(0, n_groups, group, 0)
        for g_e in range(n_groups * _PB - _PB, n_groups * _PB):
            @pl.when(valid(g_e))
            def _(g_e=g_e):
                wait_store(g_e % _PB)

    return prep(tok_t, tok_tail)


@functools.partial(jax.jit, static_argnums=(3, 4, 5))
def _emb_call(tok_lin, idx, pe_ext, N, D, S):
    n_per_w = N // _NW
    n_chunks = n_per_w // _CH
    assert n_chunks % _NBUF == 0
    pe_rows = pe_ext.shape[0]
    mesh = plsc.VectorSubcoreMesh(core_axis_name="c", subcore_axis_name="s")

    @functools.partial(
        pl.kernel,
        out_type=jax.ShapeDtypeStruct((N, _DP), jnp.float32),
        mesh=mesh,
        scratch_types=[
            pltpu.VMEM((n_per_w,), jnp.int32),
            [pltpu.VMEM((_CH, D), jnp.float32)] * _NBUF,
            pltpu.VMEM((pe_rows, D), jnp.float32),
            [pltpu.SemaphoreType.DMA] * _NBUF,
            [pltpu.SemaphoreType.DMA] * _NBUF,
        ],
        compiler_params=pltpu.CompilerParams(use_tc_tiling_on_sc=False),
    )
    def emb(tok_hbm, idx_hbm, pe_hbm, out_hbm, idx_v, bufs, pe_v, gsems, ssems):
        wid = lax.axis_index("s") * _NC + lax.axis_index("c")
        base = wid * n_per_w
        pltpu.sync_copy(idx_hbm.at[pl.ds(base, n_per_w)], idx_v)
        pltpu.sync_copy(pe_hbm, pe_v)

        def fire_gather(g, bi):
            pltpu.async_copy(
                tok_hbm.at[idx_v.at[pl.ds(g * _CH, _CH)]], bufs[bi], gsems[bi]
            )

        def wait_gather(bi):
            pltpu.make_async_copy(
                tok_hbm.at[idx_v.at[pl.ds(0, _CH)]], bufs[bi], gsems[bi]
            ).wait()

        def wait_store(bi):
            pltpu.make_async_copy(
                bufs[bi], out_hbm.at[pl.ds(0, _CH), pl.ds(0, D)], ssems[bi]
            ).wait()

        fire_gather(0, 0)
        fire_gather(1, 1)

        def process(g, bi):
            wait_gather(bi)
            start = lax.rem(base + g * _CH, S)
            buf = bufs[bi]

            @plsc.parallel_loop(0, _CH, 1, unroll=8)
            def radd(r):
                p = start + r
                for k in range(D // 16):
                    sl = pl.ds(k * 16, 16)
                    buf[r, sl] = buf[r, sl] + pe_v[p, sl]

            pltpu.async_copy(
                buf,
                out_hbm.at[pl.ds(base + g * _CH, _CH), pl.ds(0, D)],
                ssems[bi],
            )

            nb = (bi + 2) % _NBUF

            @pl.when(g >= 2)
            def _():
                wait_store(nb)

            @pl.when(g + 2 < n_chunks)
            def _():
                fire_gather(g + 2, nb)

        def group(t, _):
            for b in range(_NBUF):
                process(t * _NBUF + b, b)
            return 0

        lax.fori_loop(0, n_chunks // _NBUF, group, 0)
        wait_store(_NBUF - 2)
        wait_store(_NBUF - 1)

    return emb(tok_lin, idx, pe_ext)


def kernel(x, tok_table):
    B, S = x.shape
    V, D = tok_table.shape
    N = B * S
    idx = x.reshape(N).astype(jnp.int32) * 2
    pe = _pe_table(S, D)
    pe_ext = jnp.concatenate([pe, pe[:_CH]], axis=0)
    tok_pad = _prep_table_tc(tok_table.T, jnp.eye(D, dtype=jnp.float32))
    tok_lin = tok_pad.reshape(V * 2, D)
    out = _emb_call(tok_lin, idx, pe_ext, N, D, S)
    return out.reshape(B, S, _DP)[:, :, :D]


# speedup vs baseline: 2.5328x; 1.0510x over previous
import functools

import jax
import jax.numpy as jnp
from jax import lax
from jax.experimental import pallas as pl
from jax.experimental.pallas import tpu as pltpu
from jax.experimental.pallas import tpu_sc as plsc

_NC = 2
_NS = 16
_NW = _NC * _NS
_CH = 256
_CG = 128
_NBUF = 4
_DP = 128


_TBLK = 32768


def _prep_table_tc(tok_t, eye):
    D, V = tok_t.shape
    grid = (V + _TBLK - 1) // _TBLK

    def body(in_ref, eye_ref, out_ref):
        out_ref[:, :D] = jax.lax.dot_general(
            in_ref[...],
            eye_ref[...],
            (((0,), (0,)), ((), ())),
            preferred_element_type=jnp.float32,
        )

    return pl.pallas_call(
        body,
        grid=(grid,),
        in_specs=[
            pl.BlockSpec((D, _TBLK), lambda j: (0, j)),
            pl.BlockSpec((D, D), lambda j: (0, 0)),
        ],
        out_specs=pl.BlockSpec((_TBLK, _DP), lambda j: (j, 0)),
        out_shape=jax.ShapeDtypeStruct((V, _DP), jnp.float32),
        compiler_params=pltpu.CompilerParams(fuse_transposed_lhs_in_matmul=True),
    )(tok_t, eye)


def _pe_table(max_len, d_embed):
    pos = jnp.arange(max_len, dtype=jnp.float32)[:, None]
    i = jnp.arange(0, d_embed, 2, dtype=jnp.float32)[None, :]
    angle = pos / jnp.power(10000.0, i / d_embed)
    pe = jnp.zeros((max_len, d_embed), dtype=jnp.float32)
    pe = pe.at[:, 0::2].set(jnp.sin(angle))
    pe = pe.at[:, 1::2].set(jnp.cos(angle))
    return pe


def _prep_table(tok_t, tok_tail):
    D, V = tok_t.shape
    n_full = V // 128
    tail = V - n_full * 128
    n_blocks = n_full + (1 if tail else 0)
    per_w = (n_blocks + _NW - 1) // _NW
    vp2 = (n_full + 1) * 64
    mesh = plsc.VectorSubcoreMesh(core_axis_name="c", subcore_axis_name="s")

    _PB = 4

    @functools.partial(
        pl.kernel,
        out_type=jax.ShapeDtypeStruct((vp2, 2 * D), jnp.float32),
        mesh=mesh,
        scratch_types=[
            [pltpu.VMEM((D, 128), jnp.float32)] * _PB,
            [pltpu.VMEM((64, 2 * D), jnp.float32)] * _PB,
            [pltpu.SemaphoreType.DMA] * _PB,
            [pltpu.SemaphoreType.DMA] * _PB,
        ],
        compiler_params=pltpu.CompilerParams(needs_layout_passes=False),
    )
    def prep(tt_hbm, ttail_hbm, out_hbm, ibufs, obufs, isems, osems):
        wid = lax.axis_index("s") * _NC + lax.axis_index("c")

        def cid(g):
            return wid * per_w + g

        def valid(g):
            return (g >= 0) & (g < per_w) & (cid(g) < n_blocks)

        def fire_load(g, bi):
            @pl.when(valid(g))
            def _():
                c = cid(g)

                @pl.when(c < n_full)
                def _():
                    pltpu.async_copy(
                        tt_hbm.at[:, pl.ds(c * 128, 128)], ibufs[bi], isems[bi]
                    )

                @pl.when(c == n_full)
                def _():
                    pltpu.async_copy(ttail_hbm, ibufs[bi], isems[bi])

        def wait_load(bi):
            pltpu.make_async_copy(
                tt_hbm.at[:, pl.ds(0, 128)], ibufs[bi], isems[bi]
            ).wait()

        def wait_store(bi):
            pltpu.make_async_copy(
                obufs[bi], out_hbm.at[pl.ds(0, 64)], osems[bi]
            ).wait()

        dvs = [lax.iota(jnp.int32, 16) + (16 * k) % D for k in range(2 * D // 16)]

        def transpose_block(bi):
            ib = ibufs[bi]
            ob = obufs[bi]

            @plsc.parallel_loop(0, 64, 1, unroll=4)
            def rows(j):
                for h in range(2):
                    iv = jnp.zeros((16,), jnp.int32) + (2 * j + h)
                    for k4 in range(D // 16):
                        k = h * (D // 16) + k4
                        ob[j, pl.ds(16 * k, 16)] = plsc.load_gather(
                            ib, [dvs[k], iv]
                        )

        fire_load(0, 0)
        fire_load(1, 1)

        def step(g, bi):
            @pl.when(valid(g - _PB))
            def _():
                wait_store(bi)

            @pl.when(valid(g))
            def _():
                wait_load(bi)
                transpose_block(bi)
                pltpu.async_copy(
                    obufs[bi], out_hbm.at[pl.ds(cid(g) * 64, 64)], osems[bi]
                )

            fire_load(g + 2, (bi + 2) % _PB)

        def group(t, _):
            for b in range(_PB):
                step(t * _PB + b, b)
            return 0

        n_groups = (per_w + _PB - 1) // _PB
        lax.fori_loop(0, n_groups, group, 0)
        for g_e in range(n_groups * _PB - _PB, n_groups * _PB):
            @pl.when(valid(g_e))
            def _(g_e=g_e):
                wait_store(g_e % _PB)

    return prep(tok_t, tok_tail)


@functools.partial(jax.jit, static_argnums=(3, 4, 5))
def _emb_call(tok_lin, idx, pe_ext, N, D, S):
    n_per_w = N // _NW
    n_chunks = n_per_w // _CH
    assert n_chunks % _NBUF == 0
    pe_rows = pe_ext.shape[0]
    mesh = plsc.VectorSubcoreMesh(core_axis_name="c", subcore_axis_name="s")

    @functools.partial(
        pl.kernel,
        out_type=jax.ShapeDtypeStruct((N, _DP), jnp.float32),
        mesh=mesh,
        scratch_types=[
            pltpu.VMEM((n_per_w,), jnp.int32),
            [pltpu.VMEM((_CH, D), jnp.float32)] * _NBUF,
            pltpu.VMEM((pe_rows, D), jnp.float32),
            [pltpu.SemaphoreType.DMA] * _NBUF,
            [pltpu.SemaphoreType.DMA] * _NBUF,
        ],
        compiler_params=pltpu.CompilerParams(use_tc_tiling_on_sc=False),
    )
    def emb(tok_hbm, idx_hbm, pe_hbm, out_hbm, idx_v, bufs, pe_v, gsems, ssems):
        wid = lax.axis_index("s") * _NC + lax.axis_index("c")
        base = wid * n_per_w
        pltpu.sync_copy(idx_hbm.at[pl.ds(base, n_per_w)], idx_v)
        pltpu.sync_copy(pe_hbm, pe_v)

        def fire_gather(g, bi):
            for h in range(_CH // _CG):
                pltpu.async_copy(
                    tok_hbm.at[idx_v.at[pl.ds(g * _CH + h * _CG, _CG)]],
                    bufs[bi].at[pl.ds(h * _CG, _CG)],
                    gsems[bi],
                )

        def wait_gather(bi):
            for h in range(_CH // _CG):
                pltpu.make_async_copy(
                    tok_hbm.at[idx_v.at[pl.ds(0, _CG)]],
                    bufs[bi].at[pl.ds(h * _CG, _CG)],
                    gsems[bi],
                ).wait()

        def wait_store(bi):
            pltpu.make_async_copy(
                bufs[bi], out_hbm.at[pl.ds(0, _CH), pl.ds(0, D)], ssems[bi]
            ).wait()

        fire_gather(0, 0)
        fire_gather(1, 1)

        def process(g, bi):
            wait_gather(bi)
            start = lax.rem(base + g * _CH, S)
            buf = bufs[bi]

            @plsc.parallel_loop(0, _CH, 1, unroll=8)
            def radd(r):
                p = start + r
                for k in range(D // 16):
                    sl = pl.ds(k * 16, 16)
                    buf[r, sl] = buf[r, sl] + pe_v[p, sl]

            pltpu.async_copy(
                buf,
                out_hbm.at[pl.ds(base + g * _CH, _CH), pl.ds(0, D)],
                ssems[bi],
            )

            nb = (bi + 2) % _NBUF

            @pl.when(g >= 2)
            def _():
                wait_store(nb)

            @pl.when(g + 2 < n_chunks)
            def _():
                fire_gather(g + 2, nb)

        def group(t, _):
            for b in range(_NBUF):
                process(t * _NBUF + b, b)
            return 0

        lax.fori_loop(0, n_chunks // _NBUF, group, 0)
        wait_store(_NBUF - 2)
        wait_store(_NBUF - 1)

    return emb(tok_lin, idx, pe_ext)


def kernel(x, tok_table):
    B, S = x.shape
    V, D = tok_table.shape
    N = B * S
    idx = x.reshape(N).astype(jnp.int32) * 2
    pe = _pe_table(S, D)
    reps = -(-(S + _CH) // S)
    pe_ext = jnp.tile(pe, (reps, 1))[: S + _CH]
    tok_pad = _prep_table_tc(tok_table.T, jnp.eye(D, dtype=jnp.float32))
    tok_lin = tok_pad.reshape(V * 2, D)
    out = _emb_call(tok_lin, idx, pe_ext, N, D, S)
    return out.reshape(B, S, _DP)[:, :, :D]


# speedup vs baseline: 2.5348x; 1.0008x over previous
import functools

import jax
import jax.numpy as jnp
from jax import lax
from jax.experimental import pallas as pl
from jax.experimental.pallas import tpu as pltpu
from jax.experimental.pallas import tpu_sc as plsc

_NC = 2
_NS = 16
_NW = _NC * _NS
_CH = 256
_CG = 128
_NBUF = 4
_DP = 128


_TBLK = 32768


def _prep_table_tc(tok_t, eye):
    D, V = tok_t.shape
    grid = (V + _TBLK - 1) // _TBLK

    def body(in_ref, eye_ref, out_ref):
        out_ref[:, :D] = jax.lax.dot_general(
            in_ref[...],
            eye_ref[...],
            (((0,), (0,)), ((), ())),
            preferred_element_type=jnp.float32,
        )

    return pl.pallas_call(
        body,
        grid=(grid,),
        in_specs=[
            pl.BlockSpec((D, _TBLK), lambda j: (0, j)),
            pl.BlockSpec((D, D), lambda j: (0, 0)),
        ],
        out_specs=pl.BlockSpec((_TBLK, _DP), lambda j: (j, 0)),
        out_shape=jax.ShapeDtypeStruct((V, _DP), jnp.float32),
        compiler_params=pltpu.CompilerParams(fuse_transposed_lhs_in_matmul=True),
    )(tok_t, eye)


def _pe_table(max_len, d_embed):
    pos = jnp.arange(max_len, dtype=jnp.float32)[:, None]
    i = jnp.arange(0, d_embed, 2, dtype=jnp.float32)[None, :]
    angle = pos / jnp.power(10000.0, i / d_embed)
    pe = jnp.zeros((max_len, d_embed), dtype=jnp.float32)
    pe = pe.at[:, 0::2].set(jnp.sin(angle))
    pe = pe.at[:, 1::2].set(jnp.cos(angle))
    return pe


@functools.partial(jax.jit, static_argnums=(3, 4, 5))
def _emb_call(tok_lin, idx, pe_ext, N, D, S):
    n_per_w = N // _NW
    n_chunks = n_per_w // _CH
    assert n_chunks % _NBUF == 0
    pe_rows = pe_ext.shape[0]
    mesh = plsc.VectorSubcoreMesh(core_axis_name="c", subcore_axis_name="s")

    @functools.partial(
        pl.kernel,
        out_type=jax.ShapeDtypeStruct((N, _DP), jnp.float32),
        mesh=mesh,
        scratch_types=[
            pltpu.VMEM((n_per_w,), jnp.int32),
            [pltpu.VMEM((_CH, D), jnp.float32)] * _NBUF,
            pltpu.VMEM((pe_rows, D), jnp.float32),
            [pltpu.SemaphoreType.DMA] * _NBUF,
            [pltpu.SemaphoreType.DMA] * _NBUF,
        ],
        compiler_params=pltpu.CompilerParams(use_tc_tiling_on_sc=False),
    )
    def emb(tok_hbm, idx_hbm, pe_hbm, out_hbm, idx_v, bufs, pe_v, gsems, ssems):
        wid = lax.axis_index("s") * _NC + lax.axis_index("c")
        base = wid * n_per_w
        pltpu.sync_copy(idx_hbm.at[pl.ds(base, n_per_w)], idx_v)
        pltpu.sync_copy(pe_hbm, pe_v)

        def fire_gather(g, bi):
            for h in range(_CH // _CG):
                pltpu.async_copy(
                    tok_hbm.at[idx_v.at[pl.ds(g * _CH + h * _CG, _CG)]],
                    bufs[bi].at[pl.ds(h * _CG, _CG)],
                    gsems[bi],
                )

        def wait_gather(bi):
            for h in range(_CH // _CG):
                pltpu.make_async_copy(
                    tok_hbm.at[idx_v.at[pl.ds(0, _CG)]],
                    bufs[bi].at[pl.ds(h * _CG, _CG)],
                    gsems[bi],
                ).wait()

        def wait_store(bi):
            pltpu.make_async_copy(
                bufs[bi], out_hbm.at[pl.ds(0, _CH), pl.ds(0, D)], ssems[bi]
            ).wait()

        fire_gather(0, 0)
        fire_gather(1, 1)

        def process(g, bi):
            wait_gather(bi)
            start = lax.rem(base + g * _CH, S)
            buf = bufs[bi]

            @plsc.parallel_loop(0, _CH, 1, unroll=8)
            def radd(r):
                p = start + r
                for k in range(D // 16):
                    sl = pl.ds(k * 16, 16)
                    buf[r, sl] = buf[r, sl] + pe_v[p, sl]

            pltpu.async_copy(
                buf,
                out_hbm.at[pl.ds(base + g * _CH, _CH), pl.ds(0, D)],
                ssems[bi],
            )

            nb = (bi + 2) % _NBUF

            @pl.when(g >= 2)
            def _():
                wait_store(nb)

            @pl.when(g + 2 < n_chunks)
            def _():
                fire_gather(g + 2, nb)

        def group(t, _):
            for b in range(_NBUF):
                process(t * _NBUF + b, b)
            return 0

        lax.fori_loop(0, n_chunks // _NBUF, group, 0)
        wait_store(_NBUF - 2)
        wait_store(_NBUF - 1)

    return emb(tok_lin, idx, pe_ext)


def kernel(x, tok_table):
    B, S = x.shape
    V, D = tok_table.shape
    N = B * S
    idx = x.reshape(N).astype(jnp.int32) * 2
    pe = _pe_table(S, D)
    reps = -(-(S + _CH) // S)
    pe_ext = jnp.tile(pe, (reps, 1))[: S + _CH]
    tok_pad = _prep_table_tc(tok_table.T, jnp.eye(D, dtype=jnp.float32))
    tok_lin = tok_pad.reshape(V * 2, D)
    out = _emb_call(tok_lin, idx, pe_ext, N, D, S)
    return out.reshape(B, S, _DP)[:, :, :D]
